# baseline (speedup 1.0000x reference)
import functools

import jax
import jax.numpy as jnp
from jax import lax
from jax.experimental import pallas as pl
from jax.experimental.pallas import tpu as pltpu
from jax.experimental.pallas import tpu_sc as plsc

B = 4096
S = 50
E = 128
NTAB = 3
NC = 2
NS = 16
NW = NC * NS
BT = B // NW
GROUP = 2
IDXW = GROUP * S
ROUNDS = BT // GROUP
NCH = E // 16
HID = 256
NLAB = 10


NBUF = 4
UNROLL = 2


def _pool_body(text_r, bi_r, tri_r, w_word, w_bi, w_tri, out_hbm,
               idx_a, idx_b, bufs, acc, sems, isem):
    wid = lax.axis_index("s") * NC + lax.axis_index("c")
    rbase = wid * ROUNDS
    gbase = wid * BT

    pltpu.sync_copy(text_r.at[pl.ds(rbase, ROUNDS)], idx_a)
    pltpu.make_async_copy(bi_r.at[pl.ds(rbase, ROUNDS)], idx_b, isem).start()

    def run_table(t, tbl, idx):
        for p in range(NBUF - 1):
            pltpu.make_async_copy(
                tbl.at[idx.at[p]], bufs.at[p], sems.at[p]).start()

        def outer(jj, carry):
            for b in range(NBUF):
                j = jj * NBUF + b

                @pl.when(j + NBUF - 1 < ROUNDS)
                def _start_next():
                    pltpu.make_async_copy(
                        tbl.at[idx.at[j + NBUF - 1]],
                        bufs.at[(b + NBUF - 1) % NBUF],
                        sems.at[(b + NBUF - 1) % NBUF],
                    ).start()

                pltpu.make_async_copy(
                    tbl.at[idx.at[j]], bufs.at[b], sems.at[b]).wait()

                def inner(i, vecs, b=b):
                    new = []
                    for q in range(GROUP):
                        for c in range(NCH):
                            v = vecs[q * NCH + c]
                            for u in range(UNROLL):
                                v = v + bufs[b, q * S + UNROLL * i + u,
                                             pl.ds(c * 16, 16)]
                            new.append(v)
                    return tuple(new)

                vecs = lax.fori_loop(
                    0, S // UNROLL, inner,
                    tuple(jnp.zeros((16,), jnp.float32)
                          for _ in range(GROUP * NCH)),
                )

                for q in range(GROUP):
                    row = j * GROUP + q
                    for c in range(NCH):
                        acc[row, pl.ds(t * E + c * 16, 16)] = (
                            vecs[q * NCH + c] * (1.0 / S))
            return carry

        lax.fori_loop(0, ROUNDS // NBUF, outer, 0)

    run_table(0, w_word, idx_a)
    pltpu.make_async_copy(bi_r.at[pl.ds(rbase, ROUNDS)], idx_b, isem).wait()
    pltpu.make_async_copy(tri_r.at[pl.ds(rbase, ROUNDS)], idx_a, isem).start()
    run_table(1, w_bi, idx_b)
    pltpu.make_async_copy(tri_r.at[pl.ds(rbase, ROUNDS)], idx_a, isem).wait()
    run_table(2, w_tri, idx_a)

    pltpu.sync_copy(acc, out_hbm.at[pl.ds(gbase, BT)])


_pool = functools.partial(
    pl.kernel,
    mesh=plsc.VectorSubcoreMesh(core_axis_name="c", subcore_axis_name="s"),
    out_type=jax.ShapeDtypeStruct((B, NTAB * E), jnp.float32),
    scratch_types=[
        pltpu.VMEM((ROUNDS, IDXW), jnp.int32),
        pltpu.VMEM((ROUNDS, IDXW), jnp.int32),
        pltpu.VMEM((NBUF, IDXW, E), jnp.float32),
        pltpu.VMEM((BT, NTAB * E), jnp.float32),
        pltpu.SemaphoreType.DMA((NBUF,)),
        pltpu.SemaphoreType.DMA,
    ],
)(_pool_body)


def _mlp_body(x_ref, w1_ref, b1_ref, w2_ref, b2_ref, out_ref, prob_ref):
    x = x_ref[...]
    tdot = functools.partial(
        lax.dot_general,
        dimension_numbers=(((1,), (1,)), ((), ())),
        preferred_element_type=jnp.float32,
    )
    h = jnp.maximum(tdot(x, w1_ref[...]) + b1_ref[...][None, :], 0.0)
    logits = tdot(h, w2_ref[...]) + b2_ref[...][None, :]
    out_ref[...] = logits
    m = jnp.max(logits, axis=-1, keepdims=True)
    e = jnp.exp(logits - m)
    prob_ref[...] = e / jnp.sum(e, axis=-1, keepdims=True)


_mlp = pl.pallas_call(
    _mlp_body,
    out_shape=(
        jax.ShapeDtypeStruct((B, NLAB), jnp.float32),
        jax.ShapeDtypeStruct((B, NLAB), jnp.float32),
    ),
)


def kernel(text, bigram, trigram, W_word, W_bi, W_tri, fc1_w, fc1_b, fc2_w, fc2_b):
    text2 = text.astype(jnp.int32).reshape(-1, IDXW)
    bi2 = bigram.astype(jnp.int32).reshape(-1, IDXW)
    tri2 = trigram.astype(jnp.int32).reshape(-1, IDXW)
    pooled = _pool(text2, bi2, tri2, W_word, W_bi, W_tri)
    out, prob = _mlp(pooled, fc1_w, fc1_b, fc2_w, fc2_b)
    return (out, prob)


# speedup vs baseline: 1.0420x; 1.0420x over previous
import functools

import jax
import jax.numpy as jnp
from jax import lax
from jax.experimental import pallas as pl
from jax.experimental.pallas import tpu as pltpu
from jax.experimental.pallas import tpu_sc as plsc

B = 4096
S = 50
E = 128
NTAB = 3
NC = 2
NS = 16
NW = NC * NS
BT = B // NW
GROUP = 2
IDXW = GROUP * S
ROUNDS = BT // GROUP
NCH = E // 16
HID = 256
NLAB = 10


NBUF = 6
UNROLL = 2
TOTAL_ROUNDS = NTAB * ROUNDS


def _pool_body(text_r, bi_r, tri_r, w_word, w_bi, w_tri, out_hbm,
               idx_w, idx_bi, idx_tr, bufs, stage, sems, osems):
    wid = lax.axis_index("s") * NC + lax.axis_index("c")
    rbase = wid * ROUNDS
    gbase = wid * BT

    pltpu.sync_copy(text_r.at[pl.ds(rbase, ROUNDS)], idx_w)
    pltpu.sync_copy(bi_r.at[pl.ds(rbase, ROUNDS)], idx_bi)
    pltpu.sync_copy(tri_r.at[pl.ds(rbase, ROUNDS)], idx_tr)

    tables = ((w_word, idx_w), (w_bi, idx_bi), (w_tri, idx_tr))

    def start_round(k_tab, k_row, slot):
        tbl, idx = tables[k_tab]
        pltpu.make_async_copy(
            tbl.at[idx.at[k_row]], bufs.at[slot], sems.at[slot]).start()

    def out_copy(t, j, slot):
        return pltpu.make_async_copy(
            stage.at[pl.ds(slot * GROUP, GROUP)],
            out_hbm.at[pl.ds(gbase + j * GROUP, GROUP), pl.ds(t * E, E)],
            osems.at[slot],
        )

    for p in range(NBUF - 1):
        start_round(p % NTAB, p // NTAB, p)

    def outer(kk, carry):
        for b in range(NBUF):
            k = kk * NBUF + b
            t = b % NTAB
            j = kk * (NBUF // NTAB) + b // NTAB

            nb = (b + NBUF - 1) % NBUF
            nj = kk * (NBUF // NTAB) + (b + NBUF - 1) // NTAB

            @pl.when(k + NBUF - 1 < TOTAL_ROUNDS)
            def _start_next():
                start_round(nb % NTAB, nj, nb)

            pltpu.make_async_copy(
                tables[t][0].at[tables[t][1].at[j]],
                bufs.at[b], sems.at[b]).wait()

            def inner(i, vecs, b=b):
                new = []
                for q in range(GROUP):
                    for c in range(NCH):
                        v = vecs[q * NCH + c]
                        for u in range(UNROLL):
                            v = v + bufs[b, q * S + UNROLL * i + u,
                                         pl.ds(c * 16, 16)]
                        new.append(v)
                return tuple(new)

            vecs = lax.fori_loop(
                0, S // UNROLL, inner,
                tuple(jnp.zeros((16,), jnp.float32)
                      for _ in range(GROUP * NCH)),
            )

            @pl.when(k >= NBUF)
            def _wait_out():
                out_copy(t, j - NBUF // NTAB, b).wait()

            for q in range(GROUP):
                for c in range(NCH):
                    stage[b * GROUP + q, pl.ds(c * 16, 16)] = (
                        vecs[q * NCH + c] * (1.0 / S))
            out_copy(t, j, b).start()
        return carry

    lax.fori_loop(0, TOTAL_ROUNDS // NBUF, outer, 0)

    for b in range(NBUF):
        out_copy(b % NTAB, ROUNDS - NBUF // NTAB + b // NTAB, b).wait()


_pool = functools.partial(
    pl.kernel,
    mesh=plsc.VectorSubcoreMesh(core_axis_name="c", subcore_axis_name="s"),
    out_type=jax.ShapeDtypeStruct((B, NTAB * E), jnp.float32),
    scratch_types=[
        pltpu.VMEM((ROUNDS, IDXW), jnp.int32),
        pltpu.VMEM((ROUNDS, IDXW), jnp.int32),
        pltpu.VMEM((ROUNDS, IDXW), jnp.int32),
        pltpu.VMEM((NBUF, IDXW, E), jnp.float32),
        pltpu.VMEM((NBUF * GROUP, E), jnp.float32),
        pltpu.SemaphoreType.DMA((NBUF,)),
        pltpu.SemaphoreType.DMA((NBUF,)),
    ],
)(_pool_body)


def _mlp_body(x_ref, w1_ref, b1_ref, w2_ref, b2_ref, out_ref, prob_ref):
    x = x_ref[...]
    tdot = functools.partial(
        lax.dot_general,
        dimension_numbers=(((1,), (1,)), ((), ())),
        preferred_element_type=jnp.float32,
    )
    h = jnp.maximum(tdot(x, w1_ref[...]) + b1_ref[...][None, :], 0.0)
    logits = tdot(h, w2_ref[...]) + b2_ref[...][None, :]
    out_ref[...] = logits
    m = jnp.max(logits, axis=-1, keepdims=True)
    e = jnp.exp(logits - m)
    prob_ref[...] = e / jnp.sum(e, axis=-1, keepdims=True)


_mlp = pl.pallas_call(
    _mlp_body,
    out_shape=(
        jax.ShapeDtypeStruct((B, NLAB), jnp.float32),
        jax.ShapeDtypeStruct((B, NLAB), jnp.float32),
    ),
)


def kernel(text, bigram, trigram, W_word, W_bi, W_tri, fc1_w, fc1_b, fc2_w, fc2_b):
    text2 = text.astype(jnp.int32).reshape(-1, IDXW)
    bi2 = bigram.astype(jnp.int32).reshape(-1, IDXW)
    tri2 = trigram.astype(jnp.int32).reshape(-1, IDXW)
    pooled = _pool(text2, bi2, tri2, W_word, W_bi, W_tri)
    out, prob = _mlp(pooled, fc1_w, fc1_b, fc2_w, fc2_b)
    return (out, prob)


# speedup vs baseline: 1.0514x; 1.0090x over previous
import functools

import jax
import jax.numpy as jnp
from jax import lax
from jax.experimental import pallas as pl
from jax.experimental.pallas import tpu as pltpu
from jax.experimental.pallas import tpu_sc as plsc

B = 4096
S = 50
E = 128
NTAB = 3
NC = 2
NS = 16
NW = NC * NS
BT = B // NW
GROUP = 2
IDXW = GROUP * S
ROUNDS = BT // GROUP
NCH = E // 16
HID = 256
NLAB = 10


NBUF = 6
UNROLL = 2
TOTAL_ROUNDS = NTAB * ROUNDS


def _pool_body(text_r, bi_r, tri_r, w_word, w_bi, w_tri, out_hbm,
               idx_w, idx_bi, idx_tr, bufs, stage, sems, osems):
    wid = lax.axis_index("s") * NC + lax.axis_index("c")
    rbase = wid * ROUNDS
    gbase = wid * BT

    cp_w = pltpu.make_async_copy(text_r.at[pl.ds(rbase, ROUNDS)], idx_w,
                                 osems.at[0])
    cp_b = pltpu.make_async_copy(bi_r.at[pl.ds(rbase, ROUNDS)], idx_bi,
                                 osems.at[1])
    cp_t = pltpu.make_async_copy(tri_r.at[pl.ds(rbase, ROUNDS)], idx_tr,
                                 osems.at[2])
    cp_w.start()
    cp_b.start()
    cp_t.start()
    cp_w.wait()
    cp_b.wait()
    cp_t.wait()

    tables = ((w_word, idx_w), (w_bi, idx_bi), (w_tri, idx_tr))

    def start_round(k_tab, k_row, slot):
        tbl, idx = tables[k_tab]
        pltpu.make_async_copy(
            tbl.at[idx.at[k_row]], bufs.at[slot], sems.at[slot]).start()

    def out_copy(t, j, slot):
        return pltpu.make_async_copy(
            stage.at[pl.ds(slot * GROUP, GROUP)],
            out_hbm.at[pl.ds(gbase + j * GROUP, GROUP), pl.ds(t * E, E)],
            osems.at[slot],
        )

    for p in range(NBUF - 1):
        start_round(p % NTAB, p // NTAB, p)

    def outer(kk, carry):
        for b in range(NBUF):
            k = kk * NBUF + b
            t = b % NTAB
            j = kk * (NBUF // NTAB) + b // NTAB

            nb = (b + NBUF - 1) % NBUF
            nj = kk * (NBUF // NTAB) + (b + NBUF - 1) // NTAB

            @pl.when(k + NBUF - 1 < TOTAL_ROUNDS)
            def _start_next():
                start_round(nb % NTAB, nj, nb)

            pltpu.make_async_copy(
                tables[t][0].at[tables[t][1].at[j]],
                bufs.at[b], sems.at[b]).wait()

            def inner(i, vecs, b=b):
                new = []
                for q in range(GROUP):
                    for c in range(NCH):
                        v = vecs[q * NCH + c]
                        for u in range(UNROLL):
                            v = v + bufs[b, q * S + UNROLL * i + u,
                                         pl.ds(c * 16, 16)]
                        new.append(v)
                return tuple(new)

            vecs = lax.fori_loop(
                0, S // UNROLL, inner,
                tuple(jnp.zeros((16,), jnp.float32)
                      for _ in range(GROUP * NCH)),
            )

            @pl.when(k >= NBUF)
            def _wait_out():
                out_copy(t, j - NBUF // NTAB, b).wait()

            for q in range(GROUP):
                for c in range(NCH):
                    stage[b * GROUP + q, pl.ds(c * 16, 16)] = (
                        vecs[q * NCH + c] * (1.0 / S))
            out_copy(t, j, b).start()
        return carry

    lax.fori_loop(0, TOTAL_ROUNDS // NBUF, outer, 0)

    for b in range(NBUF):
        out_copy(b % NTAB, ROUNDS - NBUF // NTAB + b // NTAB, b).wait()


_pool = functools.partial(
    pl.kernel,
    mesh=plsc.VectorSubcoreMesh(core_axis_name="c", subcore_axis_name="s"),
    out_type=jax.ShapeDtypeStruct((B, NTAB * E), jnp.float32),
    scratch_types=[
        pltpu.VMEM((ROUNDS, IDXW), jnp.int32),
        pltpu.VMEM((ROUNDS, IDXW), jnp.int32),
        pltpu.VMEM((ROUNDS, IDXW), jnp.int32),
        pltpu.VMEM((NBUF, IDXW, E), jnp.float32),
        pltpu.VMEM((NBUF * GROUP, E), jnp.float32),
        pltpu.SemaphoreType.DMA((NBUF,)),
        pltpu.SemaphoreType.DMA((NBUF,)),
    ],
)(_pool_body)


def _mlp_body(x_ref, w1_ref, b1_ref, w2_ref, b2_ref, out_ref, prob_ref):
    x = x_ref[...]
    tdot = functools.partial(
        lax.dot_general,
        dimension_numbers=(((1,), (1,)), ((), ())),
        preferred_element_type=jnp.float32,
    )
    h = jnp.maximum(tdot(x, w1_ref[...]) + b1_ref[...][None, :], 0.0)
    logits = tdot(h, w2_ref[...]) + b2_ref[...][None, :]
    out_ref[...] = logits
    m = jnp.max(logits, axis=-1, keepdims=True)
    e = jnp.exp(logits - m)
    prob_ref[...] = e / jnp.sum(e, axis=-1, keepdims=True)


_mlp = pl.pallas_call(
    _mlp_body,
    out_shape=(
        jax.ShapeDtypeStruct((B, NLAB), jnp.float32),
        jax.ShapeDtypeStruct((B, NLAB), jnp.float32),
    ),
)


def kernel(text, bigram, trigram, W_word, W_bi, W_tri, fc1_w, fc1_b, fc2_w, fc2_b):
    text2 = text.astype(jnp.int32).reshape(-1, IDXW)
    bi2 = bigram.astype(jnp.int32).reshape(-1, IDXW)
    tri2 = trigram.astype(jnp.int32).reshape(-1, IDXW)
    pooled = _pool(text2, bi2, tri2, W_word, W_bi, W_tri)
    out, prob = _mlp(pooled, fc1_w, fc1_b, fc2_w, fc2_b)
    return (out, prob)
